# baseline (speedup 1.0000x reference)
import functools

import jax
import jax.numpy as jnp
from jax import lax
from jax.experimental import pallas as pl
from jax.experimental.pallas import tpu as pltpu
from jax.experimental.pallas import tpu_sc as plsc

_NC = 2
_NS = 16
_NW = _NC * _NS
_BLK = 384
_SUB = 128
_KB = _BLK // _SUB


def _ceil_to(a, m):
    return (a + m - 1) // m * m


def _mesh():
    return plsc.VectorSubcoreMesh(core_axis_name="c", subcore_axis_name="s",
                                  num_cores=_NC, num_subcores=_NS)


@functools.lru_cache(maxsize=None)
def _seg_sum_sc(n, n_pad, e_pad, nchunk, with_deg):
    rpt = n_pad // _NS
    wrows = e_pad // _SUB // _NW
    nblk = e_pad // (_NW * _BLK)
    zrows = 136 if rpt % 136 == 0 else 8
    zit = rpt // zrows

    out_type = [jax.ShapeDtypeStruct((_NC * n_pad, 32), jnp.float32)
                for _ in range(nchunk)]
    scratch = [
        pltpu.VMEM((_KB, 2, _SUB), jnp.int32),
        pltpu.VMEM((_KB, 2, _SUB), jnp.int32),
        [pltpu.VMEM((_KB, _SUB, 32), jnp.float32)
         for _ in range(1 if with_deg else 2)],
        pltpu.VMEM((zrows, 32), jnp.float32),
        pltpu.VMEM_SHARED((n_pad, 32), jnp.float32),
        pltpu.SemaphoreType.DMA,
        pltpu.SemaphoreType.DMA,
        pltpu.SemaphoreType.DMA,
        pltpu.SemaphoreType.DMA,
        pltpu.SemaphoreType.DMA,
    ]
    if with_deg:
        out_type.append(jax.ShapeDtypeStruct((_NC * n_pad,), jnp.float32))
        scratch += [
            pltpu.VMEM((_KB, _SUB), jnp.float32),
            pltpu.VMEM((rpt,), jnp.float32),
            pltpu.VMEM_SHARED((n_pad,), jnp.float32),
        ]

    def body(*refs):
        tables = refs[:nchunk]
        idx_c, zb_h = refs[nchunk:nchunk + 2]
        pos = nchunk + 2
        if with_deg:
            zb1_h, ones_h = refs[pos:pos + 2]
            pos += 2
        else:
            dummy_i = refs[pos]
            pos += 1
        outs = refs[pos:pos + nchunk]
        pos += nchunk
        if with_deg:
            degout = refs[pos]
            pos += 1
        (idx0, idx1, rows, zbuf, agg, gsem0, gsem1, ssem, isem0,
         isem1) = refs[pos:pos + 10]
        if with_deg:
            obuf, zbuf1, degsp = refs[pos + 10:pos + 13]

        c = lax.axis_index("c")
        s = lax.axis_index("s")
        wid = c * _NS + s

        pltpu.sync_copy(zb_h, zbuf)
        if with_deg:
            pltpu.sync_copy(ones_h, obuf)
            pltpu.sync_copy(zb1_h, zbuf1)
            pltpu.sync_copy(zbuf1, degsp.at[pl.ds(s * rpt, rpt)])

        for ci in range(nchunk):
            table = tables[ci]
            for k in range(zit):
                pltpu.sync_copy(zbuf,
                                agg.at[pl.ds(s * rpt + k * zrows, zrows)])
            plsc.subcore_barrier()

            deg_here = with_deg and ci == 0

            def fire_idx(b, buf, sem):
                rb = wid * wrows + b * _KB
                return pltpu.async_copy(idx_c.at[pl.ds(rb, _KB)], buf, sem)

            def wait_idx(buf, sem):
                pltpu.make_async_copy(idx_c.at[pl.ds(0, _KB)], buf,
                                      sem).wait()

            def fire_gather(ib, rb, sem):
                for j in range(_KB):
                    pltpu.async_copy(table.at[ib.at[j, 0]], rb.at[j], sem)

            def wait_gather(rb, sem):
                for j in range(_KB):
                    pltpu.make_async_copy(table.at[pl.ds(0, _SUB)],
                                          rb.at[j], sem).wait()

            def fire_scatter(ib, rb):
                for j in range(_KB):
                    pltpu.async_copy(rb.at[j], agg.at[ib.at[j, 1]], ssem,
                                     add=True)

            def wait_scatter(rb):
                for j in range(_KB):
                    pltpu.make_async_copy(rb.at[j],
                                          agg.at[pl.ds(0, _SUB)],
                                          ssem).wait()

            if with_deg:
                rows0 = rows[0]

                def do_block(buf):
                    fire_gather(buf, rows0, gsem0)
                    wait_gather(rows0, gsem0)
                    for j in range(_KB):
                        pltpu.sync_copy(rows0.at[j],
                                        agg.at[buf.at[j, 1]], add=True)
                    if deg_here:
                        for j in range(_KB):
                            pltpu.sync_copy(obuf.at[j],
                                            degsp.at[buf.at[j, 1]],
                                            add=True)

                fire_idx(0, idx0, isem0)

                def pair(i, carry):
                    b = 2 * i
                    wait_idx(idx0, isem0)
                    fire_idx(b + 1, idx1, isem1)
                    do_block(idx0)
                    wait_idx(idx1, isem1)
                    fire_idx(lax.min(b + 2, nblk - 1), idx0, isem0)
                    do_block(idx1)
                    return carry

                lax.fori_loop(0, nblk // 2, pair, 0)
                wait_idx(idx0, isem0)
                if nblk % 2:
                    do_block(idx0)
            else:
                rows0, rows1 = rows
                pltpu.sync_copy(dummy_i, idx1)
                fire_scatter(idx1, rows1)
                fire_idx(0, idx0, isem0)
                wait_idx(idx0, isem0)
                fire_gather(idx0, rows0, gsem0)

                def pair(i, carry):
                    b = 2 * i
                    wait_scatter(rows1)
                    fire_idx(b + 1, idx1, isem1)
                    wait_gather(rows0, gsem0)
                    fire_scatter(idx0, rows0)
                    wait_idx(idx1, isem1)
                    fire_gather(idx1, rows1, gsem1)
                    wait_scatter(rows0)
                    fire_idx(lax.min(b + 2, nblk - 1), idx0, isem0)
                    wait_gather(rows1, gsem1)
                    fire_scatter(idx1, rows1)
                    wait_idx(idx0, isem0)
                    fire_gather(idx0, rows0, gsem0)
                    return carry

                lax.fori_loop(0, nblk // 2, pair, 0)
                wait_scatter(rows1)
                wait_gather(rows0, gsem0)
            plsc.subcore_barrier()
            pltpu.sync_copy(agg.at[pl.ds(s * rpt, rpt)],
                            outs[ci].at[pl.ds(c * n_pad + s * rpt, rpt)])
        if with_deg:
            pltpu.sync_copy(degsp.at[pl.ds(s * rpt, rpt)],
                            degout.at[pl.ds(c * n_pad + s * rpt, rpt)])

    return pl.kernel(body, out_type=tuple(out_type), mesh=_mesh(),
                     scratch_types=scratch,
                     compiler_params=pltpu.CompilerParams(
                         use_tc_tiling_on_sc=False))


@functools.lru_cache(maxsize=None)
def _dense_layer_tc(n, n_pad, c_in, f_out, relu, chunk_out, bn=2048):
    grid = ((n + bn - 1) // bn,)
    f_in = 32 * c_in

    def bspec(block, imap):
        return pl.BlockSpec(block, imap)

    in_specs = (
        [bspec((2, bn, 32), lambda i: (0, i, 0)) for _ in range(c_in)]
        + [bspec((2, bn), lambda i: (0, i))]
        + [bspec((bn, 32), lambda i: (i, 0)) for _ in range(c_in)]
        + [bspec((f_in, f_out), lambda i: (0, 0)),
           bspec((f_in, f_out), lambda i: (0, 0)),
           bspec((1, f_out), lambda i: (0, 0))]
    )
    if chunk_out:
        n_out = f_out // 32
        out_specs = [bspec((bn, 32), lambda i: (i, 0)) for _ in range(n_out)]
        out_shape = [jax.ShapeDtypeStruct((n, 32), jnp.float32)
                     for _ in range(n_out)]
    else:
        out_specs = [bspec((bn, f_out), lambda i: (i, 0))]
        out_shape = [jax.ShapeDtypeStruct((n, f_out), jnp.float32)]

    def body(*refs):
        p_refs = refs[:c_in]
        deg_ref = refs[c_in]
        h_refs = refs[c_in + 1:2 * c_in + 1]
        wl_ref, wr_ref, b_ref = refs[2 * c_in + 1:2 * c_in + 4]
        out_refs = refs[2 * c_in + 4:]
        deg = deg_ref[0, :] + deg_ref[1, :]
        inv = 1.0 / jnp.maximum(deg, 1.0)
        agg = jnp.concatenate([p[0] + p[1] for p in p_refs], axis=1)
        h = jnp.concatenate([hr[...] for hr in h_refs], axis=1)
        mean = agg * inv[:, None]
        y = (jnp.dot(mean, wl_ref[...], preferred_element_type=jnp.float32)
             + jnp.dot(h, wr_ref[...], preferred_element_type=jnp.float32)
             + b_ref[...])
        if relu:
            y = jnp.maximum(y, 0.0)
        if chunk_out:
            for kc, oref in enumerate(out_refs):
                oref[...] = y[:, 32 * kc:32 * (kc + 1)]
        else:
            out_refs[0][...] = y

    return pl.pallas_call(
        body, grid=grid, in_specs=in_specs, out_specs=out_specs,
        out_shape=out_shape,
        compiler_params=pltpu.CompilerParams(
            dimension_semantics=("arbitrary",)),
    )


def kernel(x, edge_index, W1l, W1r, b1, W2l, W2r, b2, W3l, W3r, b3):
    n, f_in = x.shape
    e = edge_index.shape[1]
    n_pad = _ceil_to(n + 1, _NS * 8)
    e_pad = _ceil_to(_ceil_to(e, _NW) // _NW, _BLK) * _NW
    pad_e = e_pad - e

    src = edge_index[0]
    dst = edge_index[1]
    if pad_e:
        src = jnp.concatenate([src, jnp.zeros((pad_e,), jnp.int32)])
        dst = jnp.concatenate([dst, jnp.full((pad_e,), n, jnp.int32)])
    idx_c = jnp.stack([src.reshape(e_pad // _SUB, _SUB),
                       dst.reshape(e_pad // _SUB, _SUB)], axis=1)

    rpt = n_pad // _NS
    zrows = 136 if rpt % 136 == 0 else 8
    zb_h = jnp.zeros((zrows, 32), jnp.float32)
    zb1_h = jnp.zeros((rpt,), jnp.float32)
    ones_h = jnp.ones((_KB, _SUB), jnp.float32)

    seg = _seg_sum_sc(n, n_pad, e_pad, 1, False)
    dummy_i = jnp.full((_KB, 2, _SUB), n, jnp.int32)

    p1_flat, deg_flat = _seg_sum_sc(n, n_pad, e_pad, 1, True)(
        x, idx_c, zb_h, zb1_h, ones_h)
    p1 = p1_flat.reshape(_NC, n_pad, 32)
    deg = deg_flat.reshape(_NC, n_pad)

    b1r = b1.reshape(1, -1)
    b2r = b2.reshape(1, -1)
    b3r = b3.reshape(1, -1)

    h1 = _dense_layer_tc(n, n_pad, 1, W1l.shape[1], True, True)(
        p1, deg, x, W1l, W1r, b1r)

    p2 = [seg(hc, idx_c, zb_h, dummy_i)[0].reshape(_NC, n_pad, 32)
          for hc in h1]
    h2 = _dense_layer_tc(n, n_pad, len(h1), W2l.shape[1], True, True)(
        *p2, deg, *h1, W2l, W2r, b2r)

    p3 = [seg(hc, idx_c, zb_h, dummy_i)[0].reshape(_NC, n_pad, 32)
          for hc in h2]
    (out,) = _dense_layer_tc(n, n_pad, len(h2), W3l.shape[1], False, False)(
        *p3, deg, *h2, W3l, W3r, b3r)
    return out


# speedup vs baseline: 1.3241x; 1.3241x over previous
import functools

import jax
import jax.numpy as jnp
from jax import lax
from jax.experimental import pallas as pl
from jax.experimental.pallas import tpu as pltpu
from jax.experimental.pallas import tpu_sc as plsc

_NC = 2
_NS = 16
_NW = _NC * _NS
_BLK = 512
_SUB = 128
_KB = _BLK // _SUB


def _ceil_to(a, m):
    return (a + m - 1) // m * m


def _mesh():
    return plsc.VectorSubcoreMesh(core_axis_name="c", subcore_axis_name="s",
                                  num_cores=_NC, num_subcores=_NS)


@functools.lru_cache(maxsize=None)
def _seg_sum_sc(n, n_pad, e_pad, nchunk, with_deg):
    rpt = n_pad // _NS
    wrows = e_pad // _SUB // _NW
    nblk = e_pad // (_NW * _BLK)
    zrows = 136 if rpt % 136 == 0 else 8
    zit = rpt // zrows

    out_type = [jax.ShapeDtypeStruct((_NC * n_pad, 32), jnp.float32)
                for _ in range(nchunk)]
    scratch = [
        pltpu.VMEM((_KB, 2, _SUB), jnp.int32),
        pltpu.VMEM((_KB, 2, _SUB), jnp.int32),
        pltpu.VMEM((_KB, _SUB, 32), jnp.float32),
        pltpu.VMEM((zrows, 32), jnp.float32),
        pltpu.VMEM_SHARED((n_pad, 32), jnp.float32),
        pltpu.SemaphoreType.DMA,
        pltpu.SemaphoreType.DMA,
        pltpu.SemaphoreType.DMA,
    ]
    if with_deg:
        out_type.append(jax.ShapeDtypeStruct((_NC * n_pad,), jnp.float32))
        scratch += [
            pltpu.VMEM((_KB, _SUB), jnp.float32),
            pltpu.VMEM((rpt,), jnp.float32),
            pltpu.VMEM_SHARED((n_pad,), jnp.float32),
        ]

    def body(*refs):
        tables = refs[:nchunk]
        idx_c, zb_h = refs[nchunk:nchunk + 2]
        pos = nchunk + 2
        if with_deg:
            zb1_h, ones_h = refs[pos:pos + 2]
            pos += 2
        outs = refs[pos:pos + nchunk]
        pos += nchunk
        if with_deg:
            degout = refs[pos]
            pos += 1
        idx0, idx1, rows, zbuf, agg, gsem, isem0, isem1 = \
            refs[pos:pos + 8]
        if with_deg:
            obuf, zbuf1, degsp = refs[pos + 8:pos + 11]

        c = lax.axis_index("c")
        s = lax.axis_index("s")
        wid = c * _NS + s

        pltpu.sync_copy(zb_h, zbuf)
        if with_deg:
            pltpu.sync_copy(ones_h, obuf)
            pltpu.sync_copy(zb1_h, zbuf1)
            pltpu.sync_copy(zbuf1, degsp.at[pl.ds(s * rpt, rpt)])

        for ci in range(nchunk):
            table = tables[ci]
            for k in range(zit):
                pltpu.sync_copy(zbuf,
                                agg.at[pl.ds(s * rpt + k * zrows, zrows)])
            plsc.subcore_barrier()

            deg_here = with_deg and ci == 0

            def fire_idx(b, buf, sem):
                rb = wid * wrows + b * _KB
                return pltpu.async_copy(idx_c.at[pl.ds(rb, _KB)], buf, sem)

            def wait_idx(buf, sem):
                pltpu.make_async_copy(idx_c.at[pl.ds(0, _KB)], buf,
                                      sem).wait()

            def do_block(buf):
                gd = [pltpu.async_copy(table.at[buf.at[j, 0]],
                                       rows.at[j], gsem)
                      for j in range(_KB)]
                for d in gd:
                    d.wait()
                for j in range(_KB):
                    pltpu.sync_copy(rows.at[j], agg.at[buf.at[j, 1]],
                                    add=True)
                if deg_here:
                    for j in range(_KB):
                        pltpu.sync_copy(obuf.at[j],
                                        degsp.at[buf.at[j, 1]], add=True)

            npair = nblk // 2
            fire_idx(0, idx0, isem0)

            def pair(i, carry):
                b = 2 * i
                wait_idx(idx0, isem0)
                fire_idx(b + 1, idx1, isem1)
                do_block(idx0)
                wait_idx(idx1, isem1)
                fire_idx(lax.min(b + 2, nblk - 1), idx0, isem0)
                do_block(idx1)
                return carry

            lax.fori_loop(0, npair, pair, 0)
            wait_idx(idx0, isem0)
            if nblk % 2:
                do_block(idx0)
            plsc.subcore_barrier()
            pltpu.sync_copy(agg.at[pl.ds(s * rpt, rpt)],
                            outs[ci].at[pl.ds(c * n_pad + s * rpt, rpt)])
        if with_deg:
            pltpu.sync_copy(degsp.at[pl.ds(s * rpt, rpt)],
                            degout.at[pl.ds(c * n_pad + s * rpt, rpt)])

    return pl.kernel(body, out_type=tuple(out_type), mesh=_mesh(),
                     scratch_types=scratch,
                     compiler_params=pltpu.CompilerParams(
                         use_tc_tiling_on_sc=False))


@functools.lru_cache(maxsize=None)
def _dense_layer_tc(n, n_pad, c_in, f_out, relu, chunk_out, bn=2048):
    grid = ((n + bn - 1) // bn,)
    f_in = 32 * c_in

    def bspec(block, imap):
        return pl.BlockSpec(block, imap)

    in_specs = (
        [bspec((2, bn, 32), lambda i: (0, i, 0)) for _ in range(c_in)]
        + [bspec((2, bn), lambda i: (0, i))]
        + [bspec((bn, 32), lambda i: (i, 0)) for _ in range(c_in)]
        + [bspec((f_in, f_out), lambda i: (0, 0)),
           bspec((f_in, f_out), lambda i: (0, 0)),
           bspec((1, f_out), lambda i: (0, 0))]
    )
    if chunk_out:
        n_out = f_out // 32
        out_specs = [bspec((bn, 32), lambda i: (i, 0)) for _ in range(n_out)]
        out_shape = [jax.ShapeDtypeStruct((n, 32), jnp.float32)
                     for _ in range(n_out)]
    else:
        out_specs = [bspec((bn, f_out), lambda i: (i, 0))]
        out_shape = [jax.ShapeDtypeStruct((n, f_out), jnp.float32)]

    def body(*refs):
        p_refs = refs[:c_in]
        deg_ref = refs[c_in]
        h_refs = refs[c_in + 1:2 * c_in + 1]
        wl_ref, wr_ref, b_ref = refs[2 * c_in + 1:2 * c_in + 4]
        out_refs = refs[2 * c_in + 4:]
        deg = deg_ref[0, :] + deg_ref[1, :]
        inv = 1.0 / jnp.maximum(deg, 1.0)
        agg = jnp.concatenate([p[0] + p[1] for p in p_refs], axis=1)
        h = jnp.concatenate([hr[...] for hr in h_refs], axis=1)
        mean = agg * inv[:, None]
        y = (jnp.dot(mean, wl_ref[...], preferred_element_type=jnp.float32)
             + jnp.dot(h, wr_ref[...], preferred_element_type=jnp.float32)
             + b_ref[...])
        if relu:
            y = jnp.maximum(y, 0.0)
        if chunk_out:
            for kc, oref in enumerate(out_refs):
                oref[...] = y[:, 32 * kc:32 * (kc + 1)]
        else:
            out_refs[0][...] = y

    return pl.pallas_call(
        body, grid=grid, in_specs=in_specs, out_specs=out_specs,
        out_shape=out_shape,
        compiler_params=pltpu.CompilerParams(
            dimension_semantics=("arbitrary",)),
    )


def kernel(x, edge_index, W1l, W1r, b1, W2l, W2r, b2, W3l, W3r, b3):
    n, f_in = x.shape
    e = edge_index.shape[1]
    n_pad = _ceil_to(n + 1, _NS * 8)
    e_pad = _ceil_to(_ceil_to(e, _NW) // _NW, _BLK) * _NW
    pad_e = e_pad - e

    src = edge_index[0]
    dst = edge_index[1]
    if pad_e:
        src = jnp.concatenate([src, jnp.zeros((pad_e,), jnp.int32)])
        dst = jnp.concatenate([dst, jnp.full((pad_e,), n, jnp.int32)])
    idx_c = jnp.stack([src.reshape(e_pad // _SUB, _SUB),
                       dst.reshape(e_pad // _SUB, _SUB)], axis=1)

    rpt = n_pad // _NS
    zrows = 136 if rpt % 136 == 0 else 8
    zb_h = jnp.zeros((zrows, 32), jnp.float32)
    zb1_h = jnp.zeros((rpt,), jnp.float32)
    ones_h = jnp.ones((_KB, _SUB), jnp.float32)

    seg = _seg_sum_sc(n, n_pad, e_pad, 1, False)

    p1_flat, deg_flat = _seg_sum_sc(n, n_pad, e_pad, 1, True)(
        x, idx_c, zb_h, zb1_h, ones_h)
    p1 = p1_flat.reshape(_NC, n_pad, 32)
    deg = deg_flat.reshape(_NC, n_pad)

    b1r = b1.reshape(1, -1)
    b2r = b2.reshape(1, -1)
    b3r = b3.reshape(1, -1)

    h1 = _dense_layer_tc(n, n_pad, 1, W1l.shape[1], True, True)(
        p1, deg, x, W1l, W1r, b1r)

    p2 = [seg(hc, idx_c, zb_h)[0].reshape(_NC, n_pad, 32) for hc in h1]
    h2 = _dense_layer_tc(n, n_pad, len(h1), W2l.shape[1], True, True)(
        *p2, deg, *h1, W2l, W2r, b2r)

    p3 = [seg(hc, idx_c, zb_h)[0].reshape(_NC, n_pad, 32) for hc in h2]
    (out,) = _dense_layer_tc(n, n_pad, len(h2), W3l.shape[1], False, False)(
        *p3, deg, *h2, W3l, W3r, b3r)
    return out
